# baseline (speedup 1.0000x reference)
import functools

import jax
import jax.numpy as jnp
from jax import lax
from jax.experimental import pallas as pl
from jax.experimental.pallas import tpu as pltpu
from jax.experimental.pallas import tpu_sc as plsc


def kernel(x, pos_emb):
    seq = x.shape[1]
    d = pos_emb.shape[1]
    info = plsc.get_sparse_core_info()
    nw = info.num_cores * info.num_subcores
    rows_per = seq // nw
    mesh = plsc.VectorSubcoreMesh(core_axis_name="c", subcore_axis_name="s")

    @functools.partial(
        pl.kernel,
        mesh=mesh,
        out_type=jax.ShapeDtypeStruct((seq, d), jnp.float32),
    )
    def copy_rows(pe_hbm, out_hbm):
        wid = lax.axis_index("s") * info.num_cores + lax.axis_index("c")
        base = wid * rows_per
        pltpu.sync_copy(
            pe_hbm.at[pl.ds(base, rows_per)],
            out_hbm.at[pl.ds(base, rows_per)],
        )

    return copy_rows(pos_emb)


# speedup vs baseline: 16.7034x; 16.7034x over previous
import functools

import jax
import jax.numpy as jnp
from jax import lax
from jax.experimental import pallas as pl
from jax.experimental.pallas import tpu as pltpu
from jax.experimental.pallas import tpu_sc as plsc


def kernel(x, pos_emb):
    seq = x.shape[1]
    d = pos_emb.shape[1]
    info = plsc.get_sparse_core_info()
    nw = info.num_cores * info.num_subcores
    rows_per = seq // nw
    mesh = plsc.VectorSubcoreMesh(core_axis_name="c", subcore_axis_name="s")

    ch = 32
    nch = rows_per // ch

    @functools.partial(
        pl.kernel,
        mesh=mesh,
        out_type=jax.ShapeDtypeStruct((seq, d), jnp.float32),
        scratch_types=[
            pltpu.VMEM((2, ch, d), jnp.float32),
            pltpu.SemaphoreType.DMA((2,)),
            pltpu.SemaphoreType.DMA((2,)),
        ],
    )
    def copy_rows(pe_hbm, out_hbm, buf, in_sem, out_sem):
        wid = lax.axis_index("s") * info.num_cores + lax.axis_index("c")
        base = wid * rows_per
        pltpu.make_async_copy(
            pe_hbm.at[pl.ds(base, ch)], buf.at[0], in_sem.at[0]
        ).start()
        for g in range(nch):
            cur = g % 2
            nxt = (g + 1) % 2
            if g + 1 < nch:
                if g >= 1:
                    pltpu.make_async_copy(
                        buf.at[nxt],
                        out_hbm.at[pl.ds(base + (g - 1) * ch, ch)],
                        out_sem.at[nxt],
                    ).wait()
                pltpu.make_async_copy(
                    pe_hbm.at[pl.ds(base + (g + 1) * ch, ch)],
                    buf.at[nxt],
                    in_sem.at[nxt],
                ).start()
            pltpu.make_async_copy(
                pe_hbm.at[pl.ds(base + g * ch, ch)], buf.at[cur], in_sem.at[cur]
            ).wait()
            pltpu.make_async_copy(
                buf.at[cur],
                out_hbm.at[pl.ds(base + g * ch, ch)],
                out_sem.at[cur],
            ).start()
        for g in range(max(nch - 2, 0), nch):
            pltpu.make_async_copy(
                buf.at[g % 2],
                out_hbm.at[pl.ds(base + g * ch, ch)],
                out_sem.at[g % 2],
            ).wait()

    return copy_rows(pos_emb)


# speedup vs baseline: 43.5125x; 2.6050x over previous
import jax
import jax.numpy as jnp
from jax.experimental import pallas as pl


def _copy_body(pe_ref, o_ref):
    o_ref[...] = pe_ref[...]


def kernel(x, pos_emb):
    seq = x.shape[1]
    d = pos_emb.shape[1]
    block = 1024
    out = pl.pallas_call(
        _copy_body,
        grid=(seq // block,),
        in_specs=[pl.BlockSpec((block, d), lambda i: (i, 0))],
        out_specs=pl.BlockSpec((block, d), lambda i: (i, 0)),
        out_shape=jax.ShapeDtypeStruct((seq, d), jnp.float32),
    )(pos_emb)
    return out


# speedup vs baseline: 48.8423x; 1.1225x over previous
import jax
import jax.numpy as jnp
from jax.experimental import pallas as pl


def _copy_body(pe_ref, o_ref):
    o_ref[...] = pe_ref[...]


def kernel(x, pos_emb):
    seq = x.shape[1]
    d = pos_emb.shape[1]
    block = 2048
    out = pl.pallas_call(
        _copy_body,
        grid=(seq // block,),
        in_specs=[pl.BlockSpec((block, d), lambda i: (i, 0))],
        out_specs=pl.BlockSpec((block, d), lambda i: (i, 0)),
        out_shape=jax.ShapeDtypeStruct((seq, d), jnp.float32),
    )(pos_emb)
    return out
